# baseline (speedup 1.0000x reference)
import functools

import jax
import jax.numpy as jnp
from jax import lax
from jax.experimental import pallas as pl
from jax.experimental.pallas import tpu as pltpu
from jax.experimental.pallas import tpu_sc as plsc

_CHUNK_ROWS = 16
_NUM_BUFS = 6


def _make_copy_kernel(seq_len: int, model_dim: int):
    info = plsc.get_sparse_core_info()
    num_workers = info.num_cores * info.num_subcores
    rows_per_w = seq_len // num_workers
    n_chunks = rows_per_w // _CHUNK_ROWS

    mesh = plsc.VectorSubcoreMesh(core_axis_name="c", subcore_axis_name="s")

    @functools.partial(
        pl.kernel,
        mesh=mesh,
        out_type=jax.ShapeDtypeStruct((seq_len, model_dim), jnp.float32),
        scratch_types=(
            [pltpu.VMEM((_CHUNK_ROWS, model_dim), jnp.float32)] * _NUM_BUFS
            + [pltpu.SemaphoreType.DMA] * _NUM_BUFS
            + [pltpu.SemaphoreType.DMA] * _NUM_BUFS
        ),
    )
    def copy_k(emb_hbm, out_hbm, *scratch):
        bufs = scratch[:_NUM_BUFS]
        in_sems = scratch[_NUM_BUFS:2 * _NUM_BUFS]
        out_sems = scratch[2 * _NUM_BUFS:]
        wid = lax.axis_index("s") * info.num_cores + lax.axis_index("c")
        base = wid * rows_per_w

        def chunk(i):
            return pl.ds(base + i * _CHUNK_ROWS, _CHUNK_ROWS)

        in_cp = [None] * _NUM_BUFS
        out_cp = [None] * _NUM_BUFS
        for i in range(min(_NUM_BUFS, n_chunks)):
            in_cp[i] = pltpu.async_copy(emb_hbm.at[chunk(i)], bufs[i], in_sems[i])
        for i in range(n_chunks):
            b = i % _NUM_BUFS
            in_cp[b].wait()
            out_cp[b] = pltpu.async_copy(bufs[b], out_hbm.at[chunk(i)], out_sems[b])
            j = i + _NUM_BUFS
            if j < n_chunks:
                out_cp[b].wait()
                in_cp[b] = pltpu.async_copy(emb_hbm.at[chunk(j)], bufs[b], in_sems[b])
                out_cp[b] = None
        for cp in out_cp:
            if cp is not None:
                cp.wait()

    return copy_k


def kernel(x, emb_weight):
    seq_len = x.shape[1]
    return _make_copy_kernel(seq_len, emb_weight.shape[1])(emb_weight)
